# Initial kernel scaffold; baseline (speedup 1.0000x reference)
import jax
import jax.numpy as jnp
from jax.experimental import pallas as pl


def kernel(x, edge_index, Wl0, bl0, Wr0, gamma0, beta0, Wl1, bl1, Wr1, Wc, bc):
    raise NotImplementedError("write your pallas kernel here")


# speedup vs baseline: 4.7363x; 4.7363x over previous
import functools
import jax
import jax.numpy as jnp
from jax import lax
from jax.experimental import pallas as pl
from jax.experimental.pallas import tpu as pltpu
from jax.experimental.pallas import tpu_sc as plsc

N = 10000
E = 320000
D = 128
NP = 10240
NC = 2
NS = 16
NW = NC * NS
EPW = E // NW
C = 80
NCHUNK = EPW // C
RPT = NP // NS
BN_EPS = 1e-5

_MESH = dict(core_axis_name="c", subcore_axis_name="s",
             num_cores=NC, num_subcores=NS)


def _zero_acc(zrows_hbm, rows, acc, s):
    pltpu.sync_copy(zrows_hbm, rows)
    for j in range(RPT // C):
        pltpu.sync_copy(rows, acc.at[pl.ds(s * RPT + j * C, C)])


def _write_acc(out_hbm, rows, acc, c, s):
    for j in range(RPT // C):
        obase = pl.multiple_of(c * NP + s * RPT + j * C, 8)
        pltpu.sync_copy(acc.at[pl.ds(s * RPT + j * C, C)], rows)
        pltpu.sync_copy(rows, out_hbm.at[pl.ds(obase, C)])


@functools.lru_cache(maxsize=None)
def _make_seg_kernel():
    def body(feats_hbm, src_hbm, dst_hbm, zrows_hbm, out_hbm,
             acc, srcv, dstv, rows, sem):
        c = lax.axis_index("c")
        s = lax.axis_index("s")
        _zero_acc(zrows_hbm, rows, acc, s)
        plsc.subcore_barrier()

        base = (s * NC + c) * EPW

        def chunk(i, _):
            off = pl.multiple_of(base + i * C, 8)
            pltpu.sync_copy(src_hbm.at[pl.ds(off, C)], srcv)
            pltpu.sync_copy(dst_hbm.at[pl.ds(off, C)], dstv)
            pltpu.async_copy(feats_hbm.at[srcv], rows, sem).wait()
            pltpu.sync_copy(rows, acc.at[dstv], add=True)
            return 0

        lax.fori_loop(0, NCHUNK, chunk, 0)
        plsc.subcore_barrier()
        _write_acc(out_hbm, rows, acc, c, s)

    return pl.kernel(
        body,
        out_type=[jax.ShapeDtypeStruct((NC * NP, D), jnp.float32)],
        mesh=plsc.VectorSubcoreMesh(**_MESH),
        scratch_types=[
            pltpu.VMEM_SHARED((NP, D), jnp.float32),
            pltpu.VMEM((C,), jnp.int32),
            pltpu.VMEM((C,), jnp.int32),
            pltpu.VMEM((C, D), jnp.float32),
            pltpu.SemaphoreType.DMA,
        ],
    )


@functools.lru_cache(maxsize=None)
def _make_deg_kernel():
    def body(dst_hbm, zrows_hbm, ones_hbm, out_hbm,
             acc, dstv, rows, ones):
        c = lax.axis_index("c")
        s = lax.axis_index("s")
        _zero_acc(zrows_hbm, rows, acc, s)
        pltpu.sync_copy(ones_hbm, ones)
        plsc.subcore_barrier()

        base = (s * NC + c) * EPW

        def chunk(i, _):
            off = pl.multiple_of(base + i * C, 8)
            pltpu.sync_copy(dst_hbm.at[pl.ds(off, C)], dstv)
            pltpu.sync_copy(ones, acc.at[dstv], add=True)
            return 0

        lax.fori_loop(0, NCHUNK, chunk, 0)
        plsc.subcore_barrier()
        _write_acc(out_hbm, rows, acc, c, s)

    return pl.kernel(
        body,
        out_type=[jax.ShapeDtypeStruct((NC * NP, D), jnp.float32)],
        mesh=plsc.VectorSubcoreMesh(**_MESH),
        scratch_types=[
            pltpu.VMEM_SHARED((NP, D), jnp.float32),
            pltpu.VMEM((C,), jnp.int32),
            pltpu.VMEM((C, D), jnp.float32),
            pltpu.VMEM((C, D), jnp.float32),
        ],
    )


def _dense0_body(p_ref, dp_ref, x_ref, wl_ref, wr_ref, b_ref,
                 h_ref, dinv_ref):
    deg = dp_ref[0, :, 0] + dp_ref[1, :, 0]
    inv = 1.0 / jnp.maximum(deg, 1.0)
    mean = (p_ref[0] + p_ref[1]) * inv[:, None]
    h = (jnp.dot(mean, wl_ref[...], preferred_element_type=jnp.float32)
         + jnp.dot(x_ref[...], wr_ref[...], preferred_element_type=jnp.float32)
         + b_ref[...])
    h_ref[...] = jnp.maximum(h, 0.0)
    dinv_ref[...] = inv[:, None]


def _dense1_body(p_ref, dinv_ref, h_ref, wl_ref, wr_ref, b_ref, wc_ref,
                 bc_ref, o_ref):
    mean = (p_ref[0] + p_ref[1]) * dinv_ref[...]
    h1 = (jnp.dot(mean, wl_ref[...], preferred_element_type=jnp.float32)
          + jnp.dot(h_ref[...], wr_ref[...], preferred_element_type=jnp.float32)
          + b_ref[...])
    h1 = jnp.maximum(h1, 0.0)
    o_ref[...] = (jnp.dot(h1, wc_ref[...], preferred_element_type=jnp.float32)
                  + bc_ref[...])


BM = 1000
GRID = N // BM

_p_spec = pl.BlockSpec((NC, BM, D), lambda m: (0, m, 0))
_x_spec = pl.BlockSpec((BM, D), lambda m: (m, 0))
_w_spec = pl.BlockSpec((D, D), lambda m: (0, 0))
_b_spec = pl.BlockSpec((1, D), lambda m: (0, 0))
_s_spec = pl.BlockSpec((BM, 1), lambda m: (m, 0))

_dense0 = pl.pallas_call(
    _dense0_body,
    grid=(GRID,),
    in_specs=[_p_spec, _p_spec, _x_spec, _w_spec, _w_spec, _b_spec],
    out_specs=[_x_spec, _s_spec],
    out_shape=[jax.ShapeDtypeStruct((N, D), jnp.float32),
               jax.ShapeDtypeStruct((N, 1), jnp.float32)],
)

_dense1 = pl.pallas_call(
    _dense1_body,
    grid=(GRID,),
    in_specs=[_p_spec, _s_spec, _x_spec, _w_spec, _w_spec, _b_spec,
              pl.BlockSpec((D, 1), lambda m: (0, 0)),
              pl.BlockSpec((1, 1), lambda m: (0, 0))],
    out_specs=_s_spec,
    out_shape=jax.ShapeDtypeStruct((N, 1), jnp.float32),
)


@jax.jit
def kernel(x, edge_index, Wl0, bl0, Wr0, gamma0, beta0, Wl1, bl1, Wr1, Wc, bc):
    src = edge_index[0]
    dst = edge_index[1]
    zrows = jnp.zeros((C, D), jnp.float32)
    ones = jnp.ones((C, D), jnp.float32)

    (degp,) = _make_deg_kernel()(dst, zrows, ones)
    degp = degp.reshape(NC, NP, D)

    (p0,) = _make_seg_kernel()(x, src, dst, zrows)
    p0 = p0.reshape(NC, NP, D)

    g = gamma0 / jnp.sqrt(1.0 + BN_EPS)
    wl0 = Wl0 * g[None, :]
    wr0 = Wr0 * g[None, :]
    b0 = (bl0 * g + beta0)[None, :]

    h, dinv = _dense0(p0, degp, x, wl0, wr0, b0)

    (p1,) = _make_seg_kernel()(h, src, dst, zrows)
    p1 = p1.reshape(NC, NP, D)

    logits = _dense1(p1, dinv, h, Wl1, Wr1, bl1[None, :], Wc, bc[None, :])
    return jnp.squeeze(logits, -1)


# speedup vs baseline: 5.8414x; 1.2333x over previous
import functools
import jax
import jax.numpy as jnp
from jax import lax
from jax.experimental import pallas as pl
from jax.experimental.pallas import tpu as pltpu
from jax.experimental.pallas import tpu_sc as plsc

N = 10000
E = 320000
D = 128
NP = 10240
NC = 2
NS = 16
NW = NC * NS
EPW = E // NW
C = 80
NCHUNK = EPW // C
NOUT = (NCHUNK + 1) // 2
RPT = NP // NS
BN_EPS = 1e-5

_MESH = dict(core_axis_name="c", subcore_axis_name="s",
             num_cores=NC, num_subcores=NS)


def _zero_acc(zrows_hbm, rows, acc, s):
    pltpu.sync_copy(zrows_hbm, rows)
    for j in range(RPT // C):
        pltpu.sync_copy(rows, acc.at[pl.ds(s * RPT + j * C, C)])


def _write_acc(out_hbm, rows, acc, c, s):
    for j in range(RPT // C):
        obase = pl.multiple_of(c * NP + s * RPT + j * C, 8)
        pltpu.sync_copy(acc.at[pl.ds(s * RPT + j * C, C)], rows)
        pltpu.sync_copy(rows, out_hbm.at[pl.ds(obase, C)])


@functools.lru_cache(maxsize=None)
def _make_seg_kernel():
    def body(feats_hbm, src_hbm, dst_hbm, zrows_hbm, out_hbm,
             acc, srcv0, dstv0, rows0, srcv1, dstv1, rows1,
             gsem0, ssem0, gsem1, ssem1):
        c = lax.axis_index("c")
        s = lax.axis_index("s")
        _zero_acc(zrows_hbm, rows0, acc, s)
        plsc.subcore_barrier()

        base = (s * NC + c) * EPW
        bufs = ((srcv0, dstv0, rows0, gsem0, ssem0),
                (srcv1, dstv1, rows1, gsem1, ssem1))

        def load_idx(i, sv, dv):
            off = pl.multiple_of(base + i * C, 8)
            pltpu.sync_copy(src_hbm.at[pl.ds(off, C)], sv)
            pltpu.sync_copy(dst_hbm.at[pl.ds(off, C)], dv)

        load_idx(0, srcv0, dstv0)
        pltpu.async_copy(feats_hbm.at[srcv0], rows0, gsem0)

        def outer(g, _):
            for b in range(2):
                i = 2 * g + b
                sv, dv, rw, gs, ss = bufs[b]
                nsv, ndv, nrw, ngs, nss = bufs[1 - b]

                @pl.when(i < NCHUNK)
                def _process():
                    pltpu.make_async_copy(feats_hbm.at[sv], rw, gs).wait()
                    pltpu.async_copy(rw, acc.at[dv], ss, add=True)

                    @pl.when(i + 1 < NCHUNK)
                    def _prefetch():
                        @pl.when(i >= 1)
                        def _drain_prev():
                            pltpu.make_async_copy(nrw, acc.at[ndv],
                                                  nss).wait()
                        load_idx(i + 1, nsv, ndv)
                        pltpu.async_copy(feats_hbm.at[nsv], nrw, ngs)
            return 0

        lax.fori_loop(0, NOUT, outer, 0)
        pltpu.make_async_copy(rows0, acc.at[dstv0], ssem0).wait()
        pltpu.make_async_copy(rows1, acc.at[dstv1], ssem1).wait()
        plsc.subcore_barrier()
        _write_acc(out_hbm, rows0, acc, c, s)

    return pl.kernel(
        body,
        out_type=[jax.ShapeDtypeStruct((NC * NP, D), jnp.float32)],
        mesh=plsc.VectorSubcoreMesh(**_MESH),
        scratch_types=[
            pltpu.VMEM_SHARED((NP, D), jnp.float32),
            pltpu.VMEM((C,), jnp.int32),
            pltpu.VMEM((C,), jnp.int32),
            pltpu.VMEM((C, D), jnp.float32),
            pltpu.VMEM((C,), jnp.int32),
            pltpu.VMEM((C,), jnp.int32),
            pltpu.VMEM((C, D), jnp.float32),
            pltpu.SemaphoreType.DMA,
            pltpu.SemaphoreType.DMA,
            pltpu.SemaphoreType.DMA,
            pltpu.SemaphoreType.DMA,
        ],
    )


@functools.lru_cache(maxsize=None)
def _make_deg_kernel():
    def body(dst_hbm, zrows_hbm, ones_hbm, out_hbm,
             acc, dstv0, dstv1, rows, ones, isem0, isem1):
        c = lax.axis_index("c")
        s = lax.axis_index("s")
        _zero_acc(zrows_hbm, rows, acc, s)
        pltpu.sync_copy(ones_hbm, ones)
        plsc.subcore_barrier()

        base = (s * NC + c) * EPW
        bufs = ((dstv0, isem0), (dstv1, isem1))

        pltpu.sync_copy(dst_hbm.at[pl.ds(pl.multiple_of(base, 8), C)], dstv0)

        def outer(g, _):
            for b in range(2):
                i = 2 * g + b
                dv, _is = bufs[b]
                ndv, nis = bufs[1 - b]

                @pl.when(i < NCHUNK)
                def _process():
                    @pl.when(i + 1 < NCHUNK)
                    def _prefetch():
                        off = pl.multiple_of(base + (i + 1) * C, 8)
                        pltpu.async_copy(dst_hbm.at[pl.ds(off, C)], ndv, nis)
                    pltpu.sync_copy(ones, acc.at[dv], add=True)

                    @pl.when(i + 1 < NCHUNK)
                    def _wait_idx():
                        pltpu.make_async_copy(
                            dst_hbm.at[pl.ds(pl.multiple_of(base, 8), C)],
                            ndv, nis).wait()
            return 0

        lax.fori_loop(0, NOUT, outer, 0)
        plsc.subcore_barrier()
        _write_acc(out_hbm, rows, acc, c, s)

    return pl.kernel(
        body,
        out_type=[jax.ShapeDtypeStruct((NC * NP, D), jnp.float32)],
        mesh=plsc.VectorSubcoreMesh(**_MESH),
        scratch_types=[
            pltpu.VMEM_SHARED((NP, D), jnp.float32),
            pltpu.VMEM((C,), jnp.int32),
            pltpu.VMEM((C,), jnp.int32),
            pltpu.VMEM((C, D), jnp.float32),
            pltpu.VMEM((C, D), jnp.float32),
            pltpu.SemaphoreType.DMA,
            pltpu.SemaphoreType.DMA,
        ],
    )


def _dense0_body(p_ref, dp_ref, x_ref, wl_ref, wr_ref, b_ref,
                 h_ref, dinv_ref):
    deg = dp_ref[0, :, 0] + dp_ref[1, :, 0]
    inv = 1.0 / jnp.maximum(deg, 1.0)
    mean = (p_ref[0] + p_ref[1]) * inv[:, None]
    h = (jnp.dot(mean, wl_ref[...], preferred_element_type=jnp.float32)
         + jnp.dot(x_ref[...], wr_ref[...], preferred_element_type=jnp.float32)
         + b_ref[...])
    h_ref[...] = jnp.maximum(h, 0.0)
    dinv_ref[...] = inv[:, None]


def _dense1_body(p_ref, dinv_ref, h_ref, wl_ref, wr_ref, b_ref, wc_ref,
                 bc_ref, o_ref):
    mean = (p_ref[0] + p_ref[1]) * dinv_ref[...]
    h1 = (jnp.dot(mean, wl_ref[...], preferred_element_type=jnp.float32)
          + jnp.dot(h_ref[...], wr_ref[...], preferred_element_type=jnp.float32)
          + b_ref[...])
    h1 = jnp.maximum(h1, 0.0)
    o_ref[...] = (jnp.dot(h1, wc_ref[...], preferred_element_type=jnp.float32)
                  + bc_ref[...])


BM = 1000
GRID = N // BM

_p_spec = pl.BlockSpec((NC, BM, D), lambda m: (0, m, 0))
_x_spec = pl.BlockSpec((BM, D), lambda m: (m, 0))
_w_spec = pl.BlockSpec((D, D), lambda m: (0, 0))
_b_spec = pl.BlockSpec((1, D), lambda m: (0, 0))
_s_spec = pl.BlockSpec((BM, 1), lambda m: (m, 0))

_dense0 = pl.pallas_call(
    _dense0_body,
    grid=(GRID,),
    in_specs=[_p_spec, _p_spec, _x_spec, _w_spec, _w_spec, _b_spec],
    out_specs=[_x_spec, _s_spec],
    out_shape=[jax.ShapeDtypeStruct((N, D), jnp.float32),
               jax.ShapeDtypeStruct((N, 1), jnp.float32)],
)

_dense1 = pl.pallas_call(
    _dense1_body,
    grid=(GRID,),
    in_specs=[_p_spec, _s_spec, _x_spec, _w_spec, _w_spec, _b_spec,
              pl.BlockSpec((D, 1), lambda m: (0, 0)),
              pl.BlockSpec((1, 1), lambda m: (0, 0))],
    out_specs=_s_spec,
    out_shape=jax.ShapeDtypeStruct((N, 1), jnp.float32),
)


@jax.jit
def kernel(x, edge_index, Wl0, bl0, Wr0, gamma0, beta0, Wl1, bl1, Wr1, Wc, bc):
    src = edge_index[0]
    dst = edge_index[1]
    zrows = jnp.zeros((C, D), jnp.float32)
    ones = jnp.ones((C, D), jnp.float32)

    (degp,) = _make_deg_kernel()(dst, zrows, ones)
    degp = degp.reshape(NC, NP, D)

    (p0,) = _make_seg_kernel()(x, src, dst, zrows)
    p0 = p0.reshape(NC, NP, D)

    g = gamma0 / jnp.sqrt(1.0 + BN_EPS)
    wl0 = Wl0 * g[None, :]
    wr0 = Wr0 * g[None, :]
    b0 = (bl0 * g + beta0)[None, :]

    h, dinv = _dense0(p0, degp, x, wl0, wr0, b0)

    (p1,) = _make_seg_kernel()(h, src, dst, zrows)
    p1 = p1.reshape(NC, NP, D)

    logits = _dense1(p1, dinv, h, Wl1, Wr1, bl1[None, :], Wc, bc[None, :])
    return jnp.squeeze(logits, -1)


# speedup vs baseline: 8.4971x; 1.4546x over previous
import functools
import jax
import jax.numpy as jnp
from jax import lax
from jax.experimental import pallas as pl
from jax.experimental.pallas import tpu as pltpu
from jax.experimental.pallas import tpu_sc as plsc

N = 10000
E = 320000
D = 128
NP = 10240
NC = 2
NS = 16
NW = NC * NS
EPW = E // NW
C = 80
NCHUNK = EPW // C
NOUT = (NCHUNK + 1) // 2
RPT = NP // NS
BN_EPS = 1e-5

_MESH = dict(core_axis_name="c", subcore_axis_name="s",
             num_cores=NC, num_subcores=NS)


def _zero_acc(zrows_hbm, rows, acc, s):
    pltpu.sync_copy(zrows_hbm, rows)
    for j in range(RPT // C):
        pltpu.sync_copy(rows, acc.at[pl.ds(s * RPT + j * C, C)])


def _write_acc(out_hbm, rows, acc, c, s):
    for j in range(RPT // C):
        obase = pl.multiple_of(c * NP + s * RPT + j * C, 8)
        pltpu.sync_copy(acc.at[pl.ds(s * RPT + j * C, C)], rows)
        pltpu.sync_copy(rows, out_hbm.at[pl.ds(obase, C)])


@functools.lru_cache(maxsize=None)
def _make_seg_kernel():
    def body(feats_hbm, src_hbm, dst_hbm, zrows_hbm, out_hbm,
             acc, srcv0, dstv0, rows0, srcv1, dstv1, rows1,
             srcv2, dstv2, rows2, isem0, gsem0, ssem0,
             isem1, gsem1, ssem1, isem2, gsem2, ssem2):
        c = lax.axis_index("c")
        s = lax.axis_index("s")
        _zero_acc(zrows_hbm, rows0, acc, s)
        plsc.subcore_barrier()

        base = (s * NC + c) * EPW
        bufs = ((srcv0, dstv0, rows0, isem0, gsem0, ssem0),
                (srcv1, dstv1, rows1, isem1, gsem1, ssem1),
                (srcv2, dstv2, rows2, isem2, gsem2, ssem2))

        def idx_start(i, sv, dv, isem):
            off = pl.multiple_of(base + i * C, 8)
            pltpu.async_copy(src_hbm.at[pl.ds(off, C)], sv, isem)
            pltpu.async_copy(dst_hbm.at[pl.ds(off, C)], dv, isem)

        def idx_wait(i, sv, dv, isem):
            off = pl.multiple_of(base + i * C, 8)
            pltpu.make_async_copy(src_hbm.at[pl.ds(off, C)], sv, isem).wait()
            pltpu.make_async_copy(dst_hbm.at[pl.ds(off, C)], dv, isem).wait()

        idx_start(0, srcv0, dstv0, isem0)
        idx_wait(0, srcv0, dstv0, isem0)
        pltpu.async_copy(feats_hbm.at[srcv0], rows0, gsem0)
        idx_start(1, srcv1, dstv1, isem1)

        def outer(g, _):
            for b in range(3):
                i = 3 * g + b
                b1 = (b + 1) % 3
                b2 = (b + 2) % 3
                sv, dv, rw, _, gs, ss = bufs[b]

                @pl.when(i < NCHUNK)
                def _process():
                    pltpu.make_async_copy(feats_hbm.at[sv], rw, gs).wait()
                    pltpu.async_copy(rw, acc.at[dv], ss, add=True)

                    @pl.when(i + 1 < NCHUNK)
                    def _launch_next_gather():
                        nsv, ndv, nrw, nis, ngs, _ = bufs[b1]
                        idx_wait(i + 1, nsv, ndv, nis)
                        pltpu.async_copy(feats_hbm.at[nsv], nrw, ngs)

                    @pl.when(i + 2 < NCHUNK)
                    def _prefetch_idx():
                        psv, pdv, prw, pis, _, pss = bufs[b2]
                        @pl.when(i >= 1)
                        def _drain_prev():
                            pltpu.make_async_copy(prw, acc.at[pdv],
                                                  pss).wait()
                        idx_start(i + 2, psv, pdv, pis)
            return 0

        lax.fori_loop(0, (NCHUNK + 2) // 3, outer, 0)
        pltpu.make_async_copy(rows0, acc.at[dstv0], ssem0).wait()
        pltpu.make_async_copy(rows1, acc.at[dstv1], ssem1).wait()
        pltpu.make_async_copy(rows2, acc.at[dstv2], ssem2).wait()
        plsc.subcore_barrier()
        _write_acc(out_hbm, rows0, acc, c, s)

    return pl.kernel(
        body,
        out_type=[jax.ShapeDtypeStruct((NC * NP, D), jnp.float32)],
        mesh=plsc.VectorSubcoreMesh(**_MESH),
        scratch_types=(
            [pltpu.VMEM_SHARED((NP, D), jnp.float32)]
            + [t for _ in range(3)
               for t in (pltpu.VMEM((C,), jnp.int32),
                         pltpu.VMEM((C,), jnp.int32),
                         pltpu.VMEM((C, D), jnp.float32))]
            + [pltpu.SemaphoreType.DMA] * 9
        ),
    )


@functools.lru_cache(maxsize=None)
def _make_deg_kernel():
    def body(dst_hbm, zrows_hbm, ones_hbm, out_hbm,
             acc, dstv0, dstv1, rows, ones, isem0, isem1):
        c = lax.axis_index("c")
        s = lax.axis_index("s")
        _zero_acc(zrows_hbm, rows, acc, s)
        pltpu.sync_copy(ones_hbm, ones)
        plsc.subcore_barrier()

        base = (s * NC + c) * EPW
        bufs = ((dstv0, isem0), (dstv1, isem1))

        pltpu.sync_copy(dst_hbm.at[pl.ds(pl.multiple_of(base, 8), C)], dstv0)

        def outer(g, _):
            for b in range(2):
                i = 2 * g + b
                dv, _is = bufs[b]
                ndv, nis = bufs[1 - b]

                @pl.when(i < NCHUNK)
                def _process():
                    @pl.when(i + 1 < NCHUNK)
                    def _prefetch():
                        off = pl.multiple_of(base + (i + 1) * C, 8)
                        pltpu.async_copy(dst_hbm.at[pl.ds(off, C)], ndv, nis)
                    pltpu.sync_copy(ones, acc.at[dv], add=True)

                    @pl.when(i + 1 < NCHUNK)
                    def _wait_idx():
                        pltpu.make_async_copy(
                            dst_hbm.at[pl.ds(pl.multiple_of(base, 8), C)],
                            ndv, nis).wait()
            return 0

        lax.fori_loop(0, NOUT, outer, 0)
        plsc.subcore_barrier()
        _write_acc(out_hbm, rows, acc, c, s)

    return pl.kernel(
        body,
        out_type=[jax.ShapeDtypeStruct((NC * NP, D), jnp.float32)],
        mesh=plsc.VectorSubcoreMesh(**_MESH),
        scratch_types=[
            pltpu.VMEM_SHARED((NP, D), jnp.float32),
            pltpu.VMEM((C,), jnp.int32),
            pltpu.VMEM((C,), jnp.int32),
            pltpu.VMEM((C, D), jnp.float32),
            pltpu.VMEM((C, D), jnp.float32),
            pltpu.SemaphoreType.DMA,
            pltpu.SemaphoreType.DMA,
        ],
    )


def _dense0_body(p_ref, dp_ref, x_ref, wl_ref, wr_ref, b_ref,
                 h_ref, dinv_ref):
    deg = dp_ref[0, :, 0] + dp_ref[1, :, 0]
    inv = 1.0 / jnp.maximum(deg, 1.0)
    mean = (p_ref[0] + p_ref[1]) * inv[:, None]
    h = (jnp.dot(mean, wl_ref[...], preferred_element_type=jnp.float32)
         + jnp.dot(x_ref[...], wr_ref[...], preferred_element_type=jnp.float32)
         + b_ref[...])
    h_ref[...] = jnp.maximum(h, 0.0)
    dinv_ref[...] = inv[:, None]


def _dense1_body(p_ref, dinv_ref, h_ref, wl_ref, wr_ref, b_ref, wc_ref,
                 bc_ref, o_ref):
    mean = (p_ref[0] + p_ref[1]) * dinv_ref[...]
    h1 = (jnp.dot(mean, wl_ref[...], preferred_element_type=jnp.float32)
          + jnp.dot(h_ref[...], wr_ref[...], preferred_element_type=jnp.float32)
          + b_ref[...])
    h1 = jnp.maximum(h1, 0.0)
    o_ref[...] = (jnp.dot(h1, wc_ref[...], preferred_element_type=jnp.float32)
                  + bc_ref[...])


BM = 1000
GRID = N // BM

_p_spec = pl.BlockSpec((NC, BM, D), lambda m: (0, m, 0))
_x_spec = pl.BlockSpec((BM, D), lambda m: (m, 0))
_w_spec = pl.BlockSpec((D, D), lambda m: (0, 0))
_b_spec = pl.BlockSpec((1, D), lambda m: (0, 0))
_s_spec = pl.BlockSpec((BM, 1), lambda m: (m, 0))

_dense0 = pl.pallas_call(
    _dense0_body,
    grid=(GRID,),
    in_specs=[_p_spec, _p_spec, _x_spec, _w_spec, _w_spec, _b_spec],
    out_specs=[_x_spec, _s_spec],
    out_shape=[jax.ShapeDtypeStruct((N, D), jnp.float32),
               jax.ShapeDtypeStruct((N, 1), jnp.float32)],
)

_dense1 = pl.pallas_call(
    _dense1_body,
    grid=(GRID,),
    in_specs=[_p_spec, _s_spec, _x_spec, _w_spec, _w_spec, _b_spec,
              pl.BlockSpec((D, 1), lambda m: (0, 0)),
              pl.BlockSpec((1, 1), lambda m: (0, 0))],
    out_specs=_s_spec,
    out_shape=jax.ShapeDtypeStruct((N, 1), jnp.float32),
)


@jax.jit
def kernel(x, edge_index, Wl0, bl0, Wr0, gamma0, beta0, Wl1, bl1, Wr1, Wc, bc):
    src = edge_index[0]
    dst = edge_index[1]
    zrows = jnp.zeros((C, D), jnp.float32)
    ones = jnp.ones((C, D), jnp.float32)

    (degp,) = _make_deg_kernel()(dst, zrows, ones)
    degp = degp.reshape(NC, NP, D)

    (p0,) = _make_seg_kernel()(x, src, dst, zrows)
    p0 = p0.reshape(NC, NP, D)

    g = gamma0 / jnp.sqrt(1.0 + BN_EPS)
    wl0 = Wl0 * g[None, :]
    wr0 = Wr0 * g[None, :]
    b0 = (bl0 * g + beta0)[None, :]

    h, dinv = _dense0(p0, degp, x, wl0, wr0, b0)

    (p1,) = _make_seg_kernel()(h, src, dst, zrows)
    p1 = p1.reshape(NC, NP, D)

    logits = _dense1(p1, dinv, h, Wl1, Wr1, bl1[None, :], Wc, bc[None, :])
    return jnp.squeeze(logits, -1)


# speedup vs baseline: 8.7745x; 1.0326x over previous
import functools
import jax
import jax.numpy as jnp
from jax import lax
from jax.experimental import pallas as pl
from jax.experimental.pallas import tpu as pltpu
from jax.experimental.pallas import tpu_sc as plsc

N = 10000
E = 320000
D = 128
NP = 10240
NC = 2
NS = 16
NW = NC * NS
EPW = E // NW
C = 80
NCHUNK = EPW // C
NOUT = (NCHUNK + 1) // 2
RPT = NP // NS
BN_EPS = 1e-5

_MESH = dict(core_axis_name="c", subcore_axis_name="s",
             num_cores=NC, num_subcores=NS)


def _zero_acc_start(zrows_hbm, rows, acc, s, zsem):
    pltpu.sync_copy(zrows_hbm, rows)
    for j in range(RPT // C):
        pltpu.async_copy(rows, acc.at[pl.ds(s * RPT + j * C, C)], zsem)


def _zero_acc_drain(rows, acc, s, zsem):
    for j in range(RPT // C):
        pltpu.make_async_copy(rows, acc.at[pl.ds(s * RPT + j * C, C)],
                              zsem).wait()


def _write_acc(out_hbm, bufs, sems, acc, c, s):
    for j in range(RPT // C):
        b = j % 2
        obase = pl.multiple_of(c * NP + s * RPT + j * C, 8)
        if j >= 2:
            pltpu.make_async_copy(bufs[b], out_hbm.at[pl.ds(obase, C)],
                                  sems[b]).wait()
        pltpu.sync_copy(acc.at[pl.ds(s * RPT + j * C, C)], bufs[b])
        pltpu.async_copy(bufs[b], out_hbm.at[pl.ds(obase, C)], sems[b])
    for b in range(2):
        obase = pl.multiple_of(c * NP + s * RPT, 8)
        pltpu.make_async_copy(bufs[b], out_hbm.at[pl.ds(obase, C)],
                              sems[b]).wait()


@functools.lru_cache(maxsize=None)
def _make_seg_kernel():
    def body(feats_hbm, src_hbm, dst_hbm, zrows_hbm, out_hbm,
             acc, srcv0, dstv0, rows0, srcv1, dstv1, rows1,
             srcv2, dstv2, rows2, isem0, gsem0, ssem0,
             isem1, gsem1, ssem1, isem2, gsem2, ssem2, zsem):
        c = lax.axis_index("c")
        s = lax.axis_index("s")
        base = (s * NC + c) * EPW
        bufs = ((srcv0, dstv0, rows0, isem0, gsem0, ssem0),
                (srcv1, dstv1, rows1, isem1, gsem1, ssem1),
                (srcv2, dstv2, rows2, isem2, gsem2, ssem2))

        def idx_start(i, sv, dv, isem):
            off = pl.multiple_of(base + i * C, 8)
            pltpu.async_copy(src_hbm.at[pl.ds(off, C)], sv, isem)
            pltpu.async_copy(dst_hbm.at[pl.ds(off, C)], dv, isem)

        def idx_wait(i, sv, dv, isem):
            off = pl.multiple_of(base + i * C, 8)
            pltpu.make_async_copy(src_hbm.at[pl.ds(off, C)], sv, isem).wait()
            pltpu.make_async_copy(dst_hbm.at[pl.ds(off, C)], dv, isem).wait()

        idx_start(0, srcv0, dstv0, isem0)
        idx_start(1, srcv1, dstv1, isem1)
        _zero_acc_start(zrows_hbm, rows2, acc, s, zsem)
        idx_wait(0, srcv0, dstv0, isem0)
        pltpu.async_copy(feats_hbm.at[srcv0], rows0, gsem0)
        _zero_acc_drain(rows2, acc, s, zsem)
        plsc.subcore_barrier()

        def outer(g, _):
            for b in range(3):
                i = 3 * g + b
                b1 = (b + 1) % 3
                b2 = (b + 2) % 3
                sv, dv, rw, _, gs, ss = bufs[b]

                @pl.when(i < NCHUNK)
                def _process():
                    pltpu.make_async_copy(feats_hbm.at[sv], rw, gs).wait()
                    pltpu.async_copy(rw, acc.at[dv], ss, add=True)

                    @pl.when(i + 1 < NCHUNK)
                    def _launch_next_gather():
                        nsv, ndv, nrw, nis, ngs, _ = bufs[b1]
                        idx_wait(i + 1, nsv, ndv, nis)
                        pltpu.async_copy(feats_hbm.at[nsv], nrw, ngs)

                    @pl.when(i + 2 < NCHUNK)
                    def _prefetch_idx():
                        psv, pdv, prw, pis, _, pss = bufs[b2]
                        @pl.when(i >= 1)
                        def _drain_prev():
                            pltpu.make_async_copy(prw, acc.at[pdv],
                                                  pss).wait()
                        idx_start(i + 2, psv, pdv, pis)
            return 0

        lax.fori_loop(0, (NCHUNK + 2) // 3, outer, 0)
        pltpu.make_async_copy(rows0, acc.at[dstv0], ssem0).wait()
        pltpu.make_async_copy(rows1, acc.at[dstv1], ssem1).wait()
        pltpu.make_async_copy(rows2, acc.at[dstv2], ssem2).wait()
        plsc.subcore_barrier()
        _write_acc(out_hbm, (rows0, rows1), (isem0, isem1), acc, c, s)

    return pl.kernel(
        body,
        out_type=[jax.ShapeDtypeStruct((NC * NP, D), jnp.float32)],
        mesh=plsc.VectorSubcoreMesh(**_MESH),
        scratch_types=(
            [pltpu.VMEM_SHARED((NP, D), jnp.float32)]
            + [t for _ in range(3)
               for t in (pltpu.VMEM((C,), jnp.int32),
                         pltpu.VMEM((C,), jnp.int32),
                         pltpu.VMEM((C, D), jnp.float32))]
            + [pltpu.SemaphoreType.DMA] * 10
        ),
    )


@functools.lru_cache(maxsize=None)
def _make_deg_kernel():
    def body(dst_hbm, zrows_hbm, ones_hbm, out_hbm,
             acc, dstv0, dstv1, rows, ones,
             isem0, isem1, ssem0, ssem1, zsem):
        c = lax.axis_index("c")
        s = lax.axis_index("s")
        base = (s * NC + c) * EPW
        bufs = ((dstv0, isem0, ssem0), (dstv1, isem1, ssem1))

        pltpu.async_copy(dst_hbm.at[pl.ds(pl.multiple_of(base, 8), C)],
                         dstv0, isem0)
        pltpu.async_copy(dst_hbm.at[pl.ds(pl.multiple_of(base + C, 8), C)],
                         dstv1, isem1)
        _zero_acc_start(zrows_hbm, rows, acc, s, zsem)
        pltpu.sync_copy(ones_hbm, ones)
        _zero_acc_drain(rows, acc, s, zsem)
        plsc.subcore_barrier()

        def outer(g, _):
            for b in range(2):
                i = 2 * g + b
                dv, iss, ss = bufs[b]
                ndv, nis, nss = bufs[1 - b]

                @pl.when(i < NCHUNK)
                def _process():
                    pltpu.make_async_copy(
                        dst_hbm.at[pl.ds(pl.multiple_of(base, 8), C)],
                        dv, iss).wait()
                    pltpu.async_copy(ones, acc.at[dv], ss, add=True)

                    @pl.when(i + 2 < NCHUNK)
                    def _prefetch():
                        pltpu.make_async_copy(ones, acc.at[dv], ss).wait()
                        off = pl.multiple_of(base + (i + 2) * C, 8)
                        pltpu.async_copy(dst_hbm.at[pl.ds(off, C)], dv, iss)
            return 0

        lax.fori_loop(0, NOUT, outer, 0)
        pltpu.make_async_copy(ones, acc.at[dstv0], ssem0).wait()
        pltpu.make_async_copy(ones, acc.at[dstv1], ssem1).wait()
        plsc.subcore_barrier()
        _write_acc(out_hbm, (rows, ones), (isem0, isem1), acc, c, s)

    return pl.kernel(
        body,
        out_type=[jax.ShapeDtypeStruct((NC * NP, D), jnp.float32)],
        mesh=plsc.VectorSubcoreMesh(**_MESH),
        scratch_types=[
            pltpu.VMEM_SHARED((NP, D), jnp.float32),
            pltpu.VMEM((C,), jnp.int32),
            pltpu.VMEM((C,), jnp.int32),
            pltpu.VMEM((C, D), jnp.float32),
            pltpu.VMEM((C, D), jnp.float32),
            pltpu.SemaphoreType.DMA,
            pltpu.SemaphoreType.DMA,
            pltpu.SemaphoreType.DMA,
            pltpu.SemaphoreType.DMA,
            pltpu.SemaphoreType.DMA,
        ],
    )


def _dense0_body(p_ref, dp_ref, x_ref, wl_ref, wr_ref, b_ref,
                 h_ref, dinv_ref):
    deg = dp_ref[0, :, 0] + dp_ref[1, :, 0]
    inv = 1.0 / jnp.maximum(deg, 1.0)
    mean = (p_ref[0] + p_ref[1]) * inv[:, None]
    h = (jnp.dot(mean, wl_ref[...], preferred_element_type=jnp.float32)
         + jnp.dot(x_ref[...], wr_ref[...], preferred_element_type=jnp.float32)
         + b_ref[...])
    h_ref[...] = jnp.maximum(h, 0.0)
    dinv_ref[...] = inv[:, None]


def _dense1_body(p_ref, dinv_ref, h_ref, wl_ref, wr_ref, b_ref, wc_ref,
                 bc_ref, o_ref):
    mean = (p_ref[0] + p_ref[1]) * dinv_ref[...]
    h1 = (jnp.dot(mean, wl_ref[...], preferred_element_type=jnp.float32)
          + jnp.dot(h_ref[...], wr_ref[...], preferred_element_type=jnp.float32)
          + b_ref[...])
    h1 = jnp.maximum(h1, 0.0)
    o_ref[...] = (jnp.dot(h1, wc_ref[...], preferred_element_type=jnp.float32)
                  + bc_ref[...])


BM = 1000
GRID = N // BM

_p_spec = pl.BlockSpec((NC, BM, D), lambda m: (0, m, 0))
_x_spec = pl.BlockSpec((BM, D), lambda m: (m, 0))
_w_spec = pl.BlockSpec((D, D), lambda m: (0, 0))
_b_spec = pl.BlockSpec((1, D), lambda m: (0, 0))
_s_spec = pl.BlockSpec((BM, 1), lambda m: (m, 0))

_dense0 = pl.pallas_call(
    _dense0_body,
    grid=(GRID,),
    in_specs=[_p_spec, _p_spec, _x_spec, _w_spec, _w_spec, _b_spec],
    out_specs=[_x_spec, _s_spec],
    out_shape=[jax.ShapeDtypeStruct((N, D), jnp.float32),
               jax.ShapeDtypeStruct((N, 1), jnp.float32)],
)

_dense1 = pl.pallas_call(
    _dense1_body,
    grid=(GRID,),
    in_specs=[_p_spec, _s_spec, _x_spec, _w_spec, _w_spec, _b_spec,
              pl.BlockSpec((D, 1), lambda m: (0, 0)),
              pl.BlockSpec((1, 1), lambda m: (0, 0))],
    out_specs=_s_spec,
    out_shape=jax.ShapeDtypeStruct((N, 1), jnp.float32),
)


@jax.jit
def kernel(x, edge_index, Wl0, bl0, Wr0, gamma0, beta0, Wl1, bl1, Wr1, Wc, bc):
    src = edge_index[0]
    dst = edge_index[1]
    zrows = jnp.zeros((C, D), jnp.float32)
    ones = jnp.ones((C, D), jnp.float32)

    (degp,) = _make_deg_kernel()(dst, zrows, ones)
    degp = degp.reshape(NC, NP, D)

    (p0,) = _make_seg_kernel()(x, src, dst, zrows)
    p0 = p0.reshape(NC, NP, D)

    g = gamma0 / jnp.sqrt(1.0 + BN_EPS)
    wl0 = Wl0 * g[None, :]
    wr0 = Wr0 * g[None, :]
    b0 = (bl0 * g + beta0)[None, :]

    h, dinv = _dense0(p0, degp, x, wl0, wr0, b0)

    (p1,) = _make_seg_kernel()(h, src, dst, zrows)
    p1 = p1.reshape(NC, NP, D)

    logits = _dense1(p1, dinv, h, Wl1, Wr1, bl1[None, :], Wc, bc[None, :])
    return jnp.squeeze(logits, -1)
